# baseline (speedup 1.0000x reference)
import jax
import jax.numpy as jnp
from jax import lax
from jax.experimental import pallas as pl
from jax.experimental.pallas import tpu as pltpu
from jax.experimental.pallas import tpu_sc as plsc

N_INSTRUMENTS = 1000
REVERB_LENGTH = 24000
BATCH = 1024

NC, NS, L = 2, 16, 16
NW = NC * NS
B_PER_W = BATCH // NW
NBUF = 4


def _body(idx_hbm, table_hbm, out_hbm, idx_v, sems):
    sems = list(sems)
    wid = lax.axis_index("s") * NC + lax.axis_index("c")
    base = wid * B_PER_W

    pltpu.sync_copy(idx_hbm.at[pl.ds(base, B_PER_W)], idx_v)
    iota = lax.iota(jnp.int32, L)

    def row_id(j):
        idx16 = idx_v[pl.ds((j // L) * L, L)]
        return jnp.max(jnp.where(iota == (j % L), idx16, -1))

    def start(j):
        rid = row_id(j)
        pltpu.async_copy(table_hbm.at[pl.ds(rid, 1)],
                         out_hbm.at[pl.ds(base + j, 1)], sems[j % NBUF])

    def wait(j):
        pltpu.make_async_copy(table_hbm.at[pl.ds(0, 1)],
                              out_hbm.at[pl.ds(base + j, 1)],
                              sems[j % NBUF]).wait()

    for j in range(NBUF):
        start(j)
    for j in range(B_PER_W):
        wait(j)
        if j + NBUF < B_PER_W:
            start(j + NBUF)


@jax.jit
def _gather(idx, table):
    mesh = plsc.VectorSubcoreMesh(core_axis_name="c", subcore_axis_name="s")
    run = pl.kernel(
        _body,
        out_type=jax.ShapeDtypeStruct((BATCH, REVERB_LENGTH), jnp.float32),
        mesh=mesh,
        scratch_types=[
            pltpu.VMEM((B_PER_W,), jnp.int32),
            [pltpu.SemaphoreType.DMA for _ in range(NBUF)],
        ],
        compiler_params=pltpu.CompilerParams(needs_layout_passes=False),
    )
    return run(idx, table)


def kernel(piano_model, reverb_dict_weight):
    idx = piano_model.astype(jnp.int32)
    return _gather(idx, reverb_dict_weight)


# speedup vs baseline: 9.2770x; 9.2770x over previous
import jax
import jax.numpy as jnp
from jax import lax
from jax.experimental import pallas as pl
from jax.experimental.pallas import tpu as pltpu
from jax.experimental.pallas import tpu_sc as plsc

N_INSTRUMENTS = 1000
REVERB_LENGTH = 24000
BATCH = 1024

NC, NS, L = 2, 16, 16
NW = NC * NS
B_PER_W = BATCH // NW
PAIRS = B_PER_W // 2
NBUF = 2


def _body(idx2_hbm, table_hbm, out_hbm, idxp_v, bufs, gsems, wsems):
    bufs = list(bufs)
    gsems = list(gsems)
    wsems = list(wsems)

    wid = lax.axis_index("s") * NC + lax.axis_index("c")
    base = wid * B_PER_W

    pltpu.sync_copy(idx2_hbm.at[pl.ds(wid * PAIRS, PAIRS)], idxp_v)

    def start_gather(p, slot):
        pltpu.async_copy(table_hbm.at[idxp_v.at[p]], bufs[slot], gsems[slot])

    def write_copy(p, slot):
        return pltpu.make_async_copy(
            bufs[slot], out_hbm.at[pl.ds(base + 2 * p, 2)], wsems[slot])

    for s in range(NBUF):
        start_gather(s, s)

    for p in range(PAIRS):
        s = p % NBUF
        pltpu.make_async_copy(table_hbm.at[idxp_v.at[p]], bufs[s],
                              gsems[s]).wait()
        write_copy(p, s).start()
        if p + NBUF < PAIRS:
            write_copy(p, s).wait()
            start_gather(p + NBUF, s)

    for p in range(PAIRS - NBUF, PAIRS):
        write_copy(p, p % NBUF).wait()


@jax.jit
def _gather(idx2, table):
    mesh = plsc.VectorSubcoreMesh(core_axis_name="c", subcore_axis_name="s")
    run = pl.kernel(
        _body,
        out_type=jax.ShapeDtypeStruct((BATCH, REVERB_LENGTH), jnp.float32),
        mesh=mesh,
        scratch_types=[
            pltpu.VMEM((PAIRS, 2), jnp.int32),
            [pltpu.VMEM((2, REVERB_LENGTH), jnp.float32) for _ in range(NBUF)],
            [pltpu.SemaphoreType.DMA for _ in range(NBUF)],
            [pltpu.SemaphoreType.DMA for _ in range(NBUF)],
        ],
        compiler_params=pltpu.CompilerParams(use_tc_tiling_on_sc=False),
    )
    return run(idx2, table)


def kernel(piano_model, reverb_dict_weight):
    idx2 = piano_model.astype(jnp.int32).reshape(BATCH // 2, 2)
    return _gather(idx2, reverb_dict_weight)
